# baseline (device time: 22339 ns/iter reference)
import numpy as np

import jax
import jax.numpy as jnp
from jax import lax
from jax.experimental import pallas as pl
from jax.experimental.pallas import tpu as pltpu

N_DEV = 16
B, H, D = 8, 8, 64
PAGES_PER_DEV = 64
BLOCK = 16
T_LOCAL = PAGES_PER_DEV * BLOCK
TH = T_LOCAL * H
BH = B * H
LANES = 128

_c = np.arange(TH)
_REXP = (np.arange(PAGES_PER_DEV)[:, None] == (_c[None, :] >> 7)).astype(
    np.float32)
_BH_HEAD = (np.arange(H)[:, None] == (_c[None, :] & 7)).astype(
    np.float32)


def kernel(Q, K, V, bt, lens):
    k2 = K.reshape(TH, D)
    v2 = V.reshape(TH, D)
    lens2 = lens.reshape(B, 1)
    rexp = jnp.asarray(_REXP, dtype=jnp.bfloat16)
    bh_head = jnp.asarray(_BH_HEAD, dtype=jnp.bfloat16)

    def body(q_ref, k_ref, v_ref, bt_ref, lens_ref, rexp_ref, bhh_ref,
             out_ref, comm_ref, send_sems, recv_sems):
        my = lax.axis_index("i")
        scope = jax.named_scope

        barrier = pltpu.get_barrier_semaphore()
        for o in range(1, N_DEV):
            peer = lax.rem(my + o, N_DEV)
            pl.semaphore_signal(
                barrier, inc=1, device_id=(peer,),
                device_id_type=pl.DeviceIdType.MESH,
            )
        with scope("ph_barrier"):
            pl.semaphore_wait(barrier, N_DEV - 1)

        with scope("ph_wbuild"):
            base_f = lax.convert_element_type(my * PAGES_PER_DEV, jnp.float32)
            jl = lax.broadcasted_iota(jnp.int32, (B, PAGES_PER_DEV), 1)
            btm = jnp.where(jl < lens_ref[:, :], bt_ref[:, :], -1)
            btm_t = btm.astype(jnp.float32).T
            pidf = base_f + lax.broadcasted_iota(
                jnp.int32, (PAGES_PER_DEV, PAGES_PER_DEV), 1
            ).astype(jnp.float32)
            rows = []
            for i in range(B):
                cmp = (btm_t[:, i:i + 1] == pidf).astype(jnp.float32)
                rows.append(jnp.sum(cmp, axis=0, keepdims=True))
            w_page = jnp.concatenate(rows, axis=0)

        with scope("ph_wexpand"):
            erep = (lax.broadcasted_iota(jnp.int32, (BH, B), 0) // H
                    == lax.broadcasted_iota(jnp.int32, (BH, B), 1)
                    ).astype(jnp.float32)
            wp64 = lax.dot_general(
                erep, w_page, (((1,), (0,)), ((), ())),
                preferred_element_type=jnp.float32,
            )
            wpb = lax.dot_general(
                wp64.astype(jnp.bfloat16), rexp_ref[:, :],
                (((1,), (0,)), ((), ())),
                preferred_element_type=jnp.float32,
            )
            ahead = (lax.broadcasted_iota(jnp.int32, (BH, H), 0) % H
                     == lax.broadcasted_iota(jnp.int32, (BH, H), 1)
                     ).astype(jnp.bfloat16)
            hm = lax.dot_general(
                ahead, bhh_ref[:, :], (((1,), (0,)), ((), ())),
                preferred_element_type=jnp.float32,
            )
            wm_big = wpb * hm

        with scope("ph_qk_softmax"):
            scale = jnp.float32(D ** -0.5)
            qflat = jnp.concatenate(
                [q_ref[i, 0, :, :] for i in range(B)], axis=0
            ) * scale
            s_big = lax.dot_general(
                qflat, k_ref[:, :], (((1,), (1,)), ((), ())),
                preferred_element_type=jnp.float32,
            )
            m = jnp.max(s_big, axis=1, keepdims=True)
            p = jnp.exp(s_big - m) * wm_big
            l = jnp.sum(p, axis=1, keepdims=True)
        with scope("ph_pv"):
            o64 = lax.dot_general(
                p, v_ref[:, :], (((1,), (0,)), ((), ())),
                preferred_element_type=jnp.float32,
            )

        with scope("ph_pack"):
            comm_ref[0, :, 0:D] = o64.astype(jnp.bfloat16)
            comm_ref[0, :, D:D + 1] = m.astype(jnp.bfloat16)
            comm_ref[0, :, D + 1:D + 2] = l.astype(jnp.bfloat16)

        rdmas = {}
        with scope("ph_rdma_issue"):
            for o in range(1, N_DEV):
                target = lax.rem(my + o, N_DEV)
                slot = N_DEV - o
                rdmas[o] = pltpu.make_async_remote_copy(
                    src_ref=comm_ref.at[0],
                    dst_ref=comm_ref.at[slot],
                    send_sem=send_sems.at[o],
                    recv_sem=recv_sems.at[slot],
                    device_id=(target,),
                    device_id_type=pl.DeviceIdType.MESH,
                )
                rdmas[o].start()
        with scope("ph_rdma_wait"):
            for o in range(1, N_DEV):
                rdmas[o].wait()

        with scope("ph_merge"):
            o_all = comm_ref[:, :, 0:D].astype(jnp.float32)
            m_all = comm_ref[:, :, D:D + 1].astype(jnp.float32)
            l_all = comm_ref[:, :, D + 1:D + 2].astype(jnp.float32)
            m_g = jnp.max(m_all, axis=0)
            sc = jnp.exp(m_all - m_g[None])
            o_g = jnp.sum(o_all * sc, axis=0)
            l_g = jnp.sum(l_all * sc, axis=0)
            res = o_g / l_g

        with scope("ph_store"):
            for i in range(B):
                out_ref[i, 0, :, :] = res[i * H:(i + 1) * H, :]

    return pl.pallas_call(
        body,
        out_shape=jax.ShapeDtypeStruct((B, 1, H, D), jnp.float32),
        in_specs=[pl.BlockSpec(memory_space=pltpu.VMEM)] * 7,
        out_specs=pl.BlockSpec(memory_space=pltpu.VMEM),
        scratch_shapes=[
            pltpu.VMEM((N_DEV, BH, LANES), jnp.bfloat16),
            pltpu.SemaphoreType.DMA((N_DEV,)),
            pltpu.SemaphoreType.DMA((N_DEV,)),
        ],
        compiler_params=pltpu.CompilerParams(collective_id=0),
    )(Q, k2, v2, bt, lens2, rexp, bh_head)


# device time: 21597 ns/iter; 1.0344x vs baseline; 1.0344x over previous
import numpy as np

import jax
import jax.numpy as jnp
from jax import lax
from jax.experimental import pallas as pl
from jax.experimental.pallas import tpu as pltpu

N_DEV = 16
B, H, D = 8, 8, 64
PAGES_PER_DEV = 64
BLOCK = 16
T_LOCAL = PAGES_PER_DEV * BLOCK
TH = T_LOCAL * H
BH = B * H
LANES = 128

_c = np.arange(TH)
_REXP = (np.arange(PAGES_PER_DEV)[:, None] == (_c[None, :] >> 7)).astype(
    np.float32)
_BH_HEAD = (np.arange(H)[:, None] == (_c[None, :] & 7)).astype(
    np.float32)


def kernel(Q, K, V, bt, lens):
    k2 = K.reshape(TH, D)
    v2 = V.reshape(TH, D)
    lens2 = lens.reshape(B, 1)
    rexp = jnp.asarray(_REXP, dtype=jnp.bfloat16)
    bh_head = jnp.asarray(_BH_HEAD, dtype=jnp.bfloat16)

    def body(q_ref, k_ref, v_ref, bt_ref, lens_ref, rexp_ref, bhh_ref,
             out_ref, comm_ref, kv_vmem, kv_sems, send_sems, recv_sems):
        my = lax.axis_index("i")
        scope = jax.named_scope

        with scope("ph_kv_dma_start"):
            kcopy = pltpu.make_async_copy(k_ref, kv_vmem.at[0], kv_sems.at[0])
            vcopy = pltpu.make_async_copy(v_ref, kv_vmem.at[1], kv_sems.at[1])
            kcopy.start()
            vcopy.start()

        barrier = pltpu.get_barrier_semaphore()
        for o in range(1, N_DEV):
            peer = lax.rem(my + o, N_DEV)
            pl.semaphore_signal(
                barrier, inc=1, device_id=(peer,),
                device_id_type=pl.DeviceIdType.MESH,
            )

        with scope("ph_wbuild"):
            base_f = lax.convert_element_type(my * PAGES_PER_DEV, jnp.float32)
            jl = lax.broadcasted_iota(jnp.int32, (B, PAGES_PER_DEV), 1)
            btm = jnp.where(jl < lens_ref[:, :], bt_ref[:, :], -1)
            btm_t = btm.astype(jnp.float32).T
            pidf = base_f + lax.broadcasted_iota(
                jnp.int32, (PAGES_PER_DEV, PAGES_PER_DEV), 1
            ).astype(jnp.float32)
            rows = []
            for i in range(B):
                cmp = (btm_t[:, i:i + 1] == pidf).astype(jnp.float32)
                rows.append(jnp.sum(cmp, axis=0, keepdims=True))
            w_page = jnp.concatenate(rows, axis=0)

        with scope("ph_wexpand"):
            erep = (lax.broadcasted_iota(jnp.int32, (BH, B), 0) // H
                    == lax.broadcasted_iota(jnp.int32, (BH, B), 1)
                    ).astype(jnp.float32)
            wp64 = lax.dot_general(
                erep, w_page, (((1,), (0,)), ((), ())),
                preferred_element_type=jnp.float32,
            )
            wpb = lax.dot_general(
                wp64.astype(jnp.bfloat16), rexp_ref[:, :],
                (((1,), (0,)), ((), ())),
                preferred_element_type=jnp.float32,
            )
            ahead = (lax.broadcasted_iota(jnp.int32, (BH, H), 0) % H
                     == lax.broadcasted_iota(jnp.int32, (BH, H), 1)
                     ).astype(jnp.bfloat16)
            hm = lax.dot_general(
                ahead, bhh_ref[:, :], (((1,), (0,)), ((), ())),
                preferred_element_type=jnp.float32,
            )
            wm_big = wpb * hm

        with scope("ph_kv_dma_wait_k"):
            kcopy.wait()
        with scope("ph_qk_softmax"):
            scale = jnp.float32(D ** -0.5)
            qflat = jnp.concatenate(
                [q_ref[i, 0, :, :] for i in range(B)], axis=0
            ) * scale
            s_big = lax.dot_general(
                qflat, kv_vmem[0], (((1,), (1,)), ((), ())),
                preferred_element_type=jnp.float32,
            )
            m = jnp.max(s_big, axis=1, keepdims=True)
            p = jnp.exp(s_big - m) * wm_big
            l = jnp.sum(p, axis=1, keepdims=True)
        with scope("ph_kv_dma_wait_v"):
            vcopy.wait()
        with scope("ph_pv"):
            o64 = lax.dot_general(
                p, kv_vmem[1], (((1,), (0,)), ((), ())),
                preferred_element_type=jnp.float32,
            )

        with scope("ph_pack"):
            comm_ref[0, :, 0:D] = o64.astype(jnp.bfloat16)
            comm_ref[0, :, D:D + 1] = m.astype(jnp.bfloat16)
            comm_ref[0, :, D + 1:D + 2] = l.astype(jnp.bfloat16)

        rdmas = {}
        with scope("ph_barrier"):
            pl.semaphore_wait(barrier, N_DEV - 1)
        with scope("ph_rdma_issue"):
            for o in range(1, N_DEV):
                target = lax.rem(my + o, N_DEV)
                slot = N_DEV - o
                rdmas[o] = pltpu.make_async_remote_copy(
                    src_ref=comm_ref.at[0],
                    dst_ref=comm_ref.at[slot],
                    send_sem=send_sems.at[o],
                    recv_sem=recv_sems.at[slot],
                    device_id=(target,),
                    device_id_type=pl.DeviceIdType.MESH,
                )
                rdmas[o].start()
        with scope("ph_rdma_wait"):
            for o in range(1, N_DEV):
                rdmas[o].wait()

        with scope("ph_merge"):
            o_all = comm_ref[:, :, 0:D].astype(jnp.float32)
            m_all = comm_ref[:, :, D:D + 1].astype(jnp.float32)
            l_all = comm_ref[:, :, D + 1:D + 2].astype(jnp.float32)
            m_g = jnp.max(m_all, axis=0)
            sc = jnp.exp(m_all - m_g[None])
            o_g = jnp.sum(o_all * sc, axis=0)
            l_g = jnp.sum(l_all * sc, axis=0)
            res = o_g / l_g

        with scope("ph_store"):
            for i in range(B):
                out_ref[i, 0, :, :] = res[i * H:(i + 1) * H, :]

    return pl.pallas_call(
        body,
        out_shape=jax.ShapeDtypeStruct((B, 1, H, D), jnp.float32),
        in_specs=[
            pl.BlockSpec(memory_space=pltpu.VMEM),
            pl.BlockSpec(memory_space=pltpu.MemorySpace.HBM),
            pl.BlockSpec(memory_space=pltpu.MemorySpace.HBM),
            pl.BlockSpec(memory_space=pltpu.VMEM),
            pl.BlockSpec(memory_space=pltpu.VMEM),
            pl.BlockSpec(memory_space=pltpu.VMEM),
            pl.BlockSpec(memory_space=pltpu.VMEM),
        ],
        out_specs=pl.BlockSpec(memory_space=pltpu.VMEM),
        scratch_shapes=[
            pltpu.VMEM((N_DEV, BH, LANES), jnp.bfloat16),
            pltpu.VMEM((2, TH, D), jnp.float32),
            pltpu.SemaphoreType.DMA((2,)),
            pltpu.SemaphoreType.DMA((N_DEV,)),
            pltpu.SemaphoreType.DMA((N_DEV,)),
        ],
        compiler_params=pltpu.CompilerParams(collective_id=0),
    )(Q, k2, v2, bt, lens2, rexp, bh_head)


# device time: 17727 ns/iter; 1.2602x vs baseline; 1.2183x over previous
import numpy as np

import jax
import jax.numpy as jnp
from jax import lax
from jax.experimental import pallas as pl
from jax.experimental.pallas import tpu as pltpu

N_DEV = 16
B, H, D = 8, 8, 64
PAGES_PER_DEV = 64
BLOCK = 16
T_LOCAL = PAGES_PER_DEV * BLOCK
TH = T_LOCAL * H
BH = B * H
LANES = 128

_c = np.arange(TH)
_CONST = np.concatenate(
    [
        (np.arange(PAGES_PER_DEV)[:, None] == (_c[None, :] >> 7)),
        (np.arange(H)[:, None] == (_c[None, :] & 7)),
    ],
    axis=0,
).astype(np.float32)


def kernel(Q, K, V, bt, lens):
    hbm = pltpu.MemorySpace.HBM
    k2 = pltpu.with_memory_space_constraint(K.reshape(TH, D), hbm)
    v2 = pltpu.with_memory_space_constraint(V.reshape(TH, D), hbm)
    cmat = pltpu.with_memory_space_constraint(
        jnp.asarray(_CONST, dtype=jnp.bfloat16), hbm)
    qflat64 = Q.reshape(BH, D)
    qp = jnp.concatenate(
        [
            jnp.pad(qflat64, ((0, 0), (0, LANES - D))),
            jnp.pad(bt.astype(jnp.float32), ((0, 0), (0, LANES - D))),
            jnp.pad(
                jnp.broadcast_to(
                    lens.astype(jnp.float32)[:, None], (B, PAGES_PER_DEV)),
                ((0, 0), (0, LANES - D))),
        ],
        axis=0,
    )

    def body(qp_ref, k_ref, v_ref, c_ref, out_ref,
             comm_ref, kv_vmem, c_vmem, dma_sems, send_sems, recv_sems):
        my = lax.axis_index("i")
        scope = jax.named_scope

        with scope("ph_dma_start"):
            kcopy = pltpu.make_async_copy(k_ref, kv_vmem.at[0], dma_sems.at[0])
            vcopy = pltpu.make_async_copy(v_ref, kv_vmem.at[1], dma_sems.at[1])
            ccopy = pltpu.make_async_copy(c_ref, c_vmem, dma_sems.at[2])
            kcopy.start()
            vcopy.start()
            ccopy.start()

        barrier = pltpu.get_barrier_semaphore()
        for o in range(1, N_DEV):
            peer = lax.rem(my + o, N_DEV)
            pl.semaphore_signal(
                barrier, inc=1, device_id=(peer,),
                device_id_type=pl.DeviceIdType.MESH,
            )

        with scope("ph_wbuild"):
            base_f = lax.convert_element_type(my * PAGES_PER_DEV, jnp.float32)
            btf = qp_ref[BH:BH + B, 0:PAGES_PER_DEV]
            lensf = qp_ref[BH + B:BH + 2 * B, 0:PAGES_PER_DEV]
            jlf = lax.broadcasted_iota(
                jnp.int32, (B, PAGES_PER_DEV), 1).astype(jnp.float32)
            btm = jnp.where(jlf < lensf, btf, -1.0)
            btm_t = btm.T
            pidf = base_f + lax.broadcasted_iota(
                jnp.int32, (PAGES_PER_DEV, PAGES_PER_DEV), 1
            ).astype(jnp.float32)
            rows = []
            for i in range(B):
                cmp = (btm_t[:, i:i + 1] == pidf).astype(jnp.float32)
                rows.append(jnp.sum(cmp, axis=0, keepdims=True))
            w_page = jnp.concatenate(rows, axis=0)

        with scope("ph_cdma_wait"):
            ccopy.wait()
        with scope("ph_wexpand"):
            erep = (lax.broadcasted_iota(jnp.int32, (BH, B), 0) // H
                    == lax.broadcasted_iota(jnp.int32, (BH, B), 1)
                    ).astype(jnp.float32)
            wp64 = lax.dot_general(
                erep, w_page, (((1,), (0,)), ((), ())),
                preferred_element_type=jnp.float32,
            )
            wpb = lax.dot_general(
                wp64.astype(jnp.bfloat16), c_vmem[0:PAGES_PER_DEV, :],
                (((1,), (0,)), ((), ())),
                preferred_element_type=jnp.float32,
            )
            ahead = (lax.broadcasted_iota(jnp.int32, (BH, H), 0) % H
                     == lax.broadcasted_iota(jnp.int32, (BH, H), 1)
                     ).astype(jnp.bfloat16)
            hm = lax.dot_general(
                ahead, c_vmem[PAGES_PER_DEV:PAGES_PER_DEV + H, :],
                (((1,), (0,)), ((), ())),
                preferred_element_type=jnp.float32,
            )
            wm_big = wpb * hm

        with scope("ph_kdma_wait"):
            kcopy.wait()
        with scope("ph_qk_softmax"):
            scale = jnp.float32(D ** -0.5)
            qflat = qp_ref[0:BH, 0:D] * scale
            s_big = lax.dot_general(
                qflat, kv_vmem[0], (((1,), (1,)), ((), ())),
                preferred_element_type=jnp.float32,
            )
            m = jnp.max(s_big, axis=1, keepdims=True)
            p = jnp.exp(s_big - m) * wm_big
            l = jnp.sum(p, axis=1, keepdims=True)
        with scope("ph_vdma_wait"):
            vcopy.wait()
        with scope("ph_pv"):
            o64 = lax.dot_general(
                p, kv_vmem[1], (((1,), (0,)), ((), ())),
                preferred_element_type=jnp.float32,
            )

        with scope("ph_pack"):
            comm_ref[0, :, 0:D] = o64.astype(jnp.bfloat16)
            comm_ref[0, :, D:D + 1] = m.astype(jnp.bfloat16)
            comm_ref[0, :, D + 1:D + 2] = l.astype(jnp.bfloat16)

        rdmas = {}
        with scope("ph_barrier"):
            pl.semaphore_wait(barrier, N_DEV - 1)
        with scope("ph_rdma_issue"):
            for o in range(1, N_DEV):
                target = lax.rem(my + o, N_DEV)
                slot = N_DEV - o
                rdmas[o] = pltpu.make_async_remote_copy(
                    src_ref=comm_ref.at[0],
                    dst_ref=comm_ref.at[slot],
                    send_sem=send_sems.at[o],
                    recv_sem=recv_sems.at[slot],
                    device_id=(target,),
                    device_id_type=pl.DeviceIdType.MESH,
                )
                rdmas[o].start()
        with scope("ph_rdma_wait"):
            for o in range(1, N_DEV):
                rdmas[o].wait()

        with scope("ph_merge"):
            o_all = comm_ref[:, :, 0:D].astype(jnp.float32)
            m_all = comm_ref[:, :, D:D + 1].astype(jnp.float32)
            l_all = comm_ref[:, :, D + 1:D + 2].astype(jnp.float32)
            m_g = jnp.max(m_all, axis=0)
            sc = jnp.exp(m_all - m_g[None])
            o_g = jnp.sum(o_all * sc, axis=0)
            l_g = jnp.sum(l_all * sc, axis=0)
            res = o_g / l_g

        with scope("ph_store"):
            for i in range(B):
                out_ref[i, 0, :, :] = res[i * H:(i + 1) * H, :]

    return pl.pallas_call(
        body,
        out_shape=jax.ShapeDtypeStruct((B, 1, H, D), jnp.float32),
        in_specs=[
            pl.BlockSpec(memory_space=pltpu.VMEM),
            pl.BlockSpec(memory_space=pltpu.MemorySpace.HBM),
            pl.BlockSpec(memory_space=pltpu.MemorySpace.HBM),
            pl.BlockSpec(memory_space=pltpu.MemorySpace.HBM),
        ],
        out_specs=pl.BlockSpec(memory_space=pltpu.VMEM),
        scratch_shapes=[
            pltpu.VMEM((N_DEV, BH, LANES), jnp.bfloat16),
            pltpu.VMEM((2, TH, D), jnp.float32),
            pltpu.VMEM((BH + H, TH), jnp.bfloat16),
            pltpu.SemaphoreType.DMA((3,)),
            pltpu.SemaphoreType.DMA((N_DEV,)),
            pltpu.SemaphoreType.DMA((N_DEV,)),
        ],
        compiler_params=pltpu.CompilerParams(collective_id=0),
    )(qp, k2, v2, cmat)


# device time: 15828 ns/iter; 1.4114x vs baseline; 1.1200x over previous
import numpy as np

import jax
import jax.numpy as jnp
from jax import lax
from jax.experimental import pallas as pl
from jax.experimental.pallas import tpu as pltpu

N_DEV = 16
B, H, D = 8, 8, 64
PAGES_PER_DEV = 64
BLOCK = 16
T_LOCAL = PAGES_PER_DEV * BLOCK
TH = T_LOCAL * H
BH = B * H
LANES = 128

_c = np.arange(TH)
_CONST = np.concatenate(
    [
        (np.arange(PAGES_PER_DEV)[:, None] == (_c[None, :] >> 7)),
        (np.arange(H)[:, None] == (_c[None, :] & 7)),
    ],
    axis=0,
).astype(np.float32)


def kernel(Q, K, V, bt, lens):
    hbm = pltpu.MemorySpace.HBM
    k2 = pltpu.with_memory_space_constraint(K.reshape(TH, D), hbm)
    v2 = pltpu.with_memory_space_constraint(V.reshape(TH, D), hbm)
    cmat = pltpu.with_memory_space_constraint(
        jnp.asarray(_CONST, dtype=jnp.bfloat16), hbm)
    qflat64 = Q.reshape(BH, D)
    qp = jnp.concatenate(
        [
            jnp.pad(qflat64, ((0, 0), (0, LANES - D))),
            jnp.pad(
                lax.bitcast_convert_type(bt, jnp.float32),
                ((0, 0), (0, LANES - D))),
            jnp.pad(
                jnp.broadcast_to(
                    lax.bitcast_convert_type(lens, jnp.float32)[:, None],
                    (B, PAGES_PER_DEV)),
                ((0, 0), (0, LANES - D))),
        ],
        axis=0,
    )

    def body(qp_ref, k_ref, v_ref, c_ref, out_ref,
             comm_ref, kv_vmem, c_vmem, dma_sems, send_sems, recv_sems):
        my = lax.axis_index("i")
        scope = jax.named_scope

        with scope("ph_dma_start"):
            kcopy = pltpu.make_async_copy(k_ref, kv_vmem.at[0], dma_sems.at[0])
            vcopy = pltpu.make_async_copy(v_ref, kv_vmem.at[1], dma_sems.at[1])
            ccopy = pltpu.make_async_copy(c_ref, c_vmem, dma_sems.at[2])
            ccopy.start()
            kcopy.start()
            vcopy.start()

        barrier = pltpu.get_barrier_semaphore()
        for o in range(1, N_DEV):
            peer = lax.rem(my + o, N_DEV)
            pl.semaphore_signal(
                barrier, inc=1, device_id=(peer,),
                device_id_type=pl.DeviceIdType.MESH,
            )

        with scope("ph_wbuild"):
            base_f = lax.convert_element_type(my * PAGES_PER_DEV, jnp.float32)
            bti = lax.bitcast_convert_type(
                qp_ref[BH:BH + B, 0:PAGES_PER_DEV], jnp.int32)
            lensi = lax.bitcast_convert_type(
                qp_ref[BH + B:BH + 2 * B, 0:PAGES_PER_DEV], jnp.int32)
            jl = lax.broadcasted_iota(jnp.int32, (B, PAGES_PER_DEV), 1)
            btm = jnp.where(jl < lensi, bti, -1).astype(jnp.float32)
            btm_t = btm.T
            pidf = base_f + lax.broadcasted_iota(
                jnp.int32, (PAGES_PER_DEV, PAGES_PER_DEV), 1
            ).astype(jnp.float32)
            rows = []
            for i in range(B):
                cmp = (btm_t[:, i:i + 1] == pidf).astype(jnp.float32)
                rows.append(jnp.sum(cmp, axis=0, keepdims=True))
            w_page = jnp.concatenate(rows, axis=0)

        with scope("ph_cdma_wait"):
            ccopy.wait()
        with scope("ph_wexpand"):
            erep = (lax.broadcasted_iota(jnp.int32, (BH, B), 0) // H
                    == lax.broadcasted_iota(jnp.int32, (BH, B), 1)
                    ).astype(jnp.float32)
            wp64 = lax.dot_general(
                erep, w_page, (((1,), (0,)), ((), ())),
                preferred_element_type=jnp.float32,
            )
            wpb = lax.dot_general(
                wp64.astype(jnp.bfloat16), c_vmem[0:PAGES_PER_DEV, :],
                (((1,), (0,)), ((), ())),
                preferred_element_type=jnp.float32,
            )
            ahead = (lax.broadcasted_iota(jnp.int32, (BH, H), 0) % H
                     == lax.broadcasted_iota(jnp.int32, (BH, H), 1)
                     ).astype(jnp.bfloat16)
            hm = lax.dot_general(
                ahead, c_vmem[PAGES_PER_DEV:PAGES_PER_DEV + H, :],
                (((1,), (0,)), ((), ())),
                preferred_element_type=jnp.float32,
            )
            wm_big = wpb * hm

        with scope("ph_kdma_wait"):
            kcopy.wait()
        with scope("ph_qk_softmax"):
            scale = jnp.float32(D ** -0.5)
            qflat = qp_ref[0:BH, 0:D] * scale
            s_big = lax.dot_general(
                qflat, kv_vmem[0], (((1,), (1,)), ((), ())),
                preferred_element_type=jnp.float32,
            )
            m = jnp.max(s_big, axis=1, keepdims=True)
            p = jnp.exp(s_big - m) * wm_big
            l = jnp.sum(p, axis=1, keepdims=True)
        with scope("ph_vdma_wait"):
            vcopy.wait()
        with scope("ph_pv"):
            o64 = lax.dot_general(
                p, kv_vmem[1], (((1,), (0,)), ((), ())),
                preferred_element_type=jnp.float32,
            )

        with scope("ph_pack"):
            comm_ref[0, :, 0:D] = o64.astype(jnp.bfloat16)
            comm_ref[0, :, D:D + 1] = m.astype(jnp.bfloat16)
            comm_ref[0, :, D + 1:D + 2] = l.astype(jnp.bfloat16)

        rdmas = {}
        with scope("ph_barrier"):
            pl.semaphore_wait(barrier, N_DEV - 1)
        with scope("ph_rdma_issue"):
            for o in range(1, N_DEV):
                target = lax.rem(my + o, N_DEV)
                slot = N_DEV - o
                rdmas[o] = pltpu.make_async_remote_copy(
                    src_ref=comm_ref.at[0],
                    dst_ref=comm_ref.at[slot],
                    send_sem=send_sems.at[o],
                    recv_sem=recv_sems.at[slot],
                    device_id=(target,),
                    device_id_type=pl.DeviceIdType.MESH,
                )
                rdmas[o].start()
        with scope("ph_rdma_wait"):
            for o in range(1, N_DEV):
                rdmas[o].wait()

        with scope("ph_merge"):
            o_all = comm_ref[:, :, 0:D].astype(jnp.float32)
            m_all = comm_ref[:, :, D:D + 1].astype(jnp.float32)
            l_all = comm_ref[:, :, D + 1:D + 2].astype(jnp.float32)
            m_g = jnp.max(m_all, axis=0)
            sc = jnp.exp(m_all - m_g[None])
            o_g = jnp.sum(o_all * sc, axis=0)
            l_g = jnp.sum(l_all * sc, axis=0)
            res = o_g / l_g

        with scope("ph_store"):
            for i in range(B):
                out_ref[i, 0, :, :] = res[i * H:(i + 1) * H, :]

    return pl.pallas_call(
        body,
        out_shape=jax.ShapeDtypeStruct((B, 1, H, D), jnp.float32),
        in_specs=[
            pl.BlockSpec(memory_space=pltpu.VMEM),
            pl.BlockSpec(memory_space=pltpu.MemorySpace.HBM),
            pl.BlockSpec(memory_space=pltpu.MemorySpace.HBM),
            pl.BlockSpec(memory_space=pltpu.MemorySpace.HBM),
        ],
        out_specs=pl.BlockSpec(memory_space=pltpu.VMEM),
        scratch_shapes=[
            pltpu.VMEM((N_DEV, BH, LANES), jnp.bfloat16),
            pltpu.VMEM((2, TH, D), jnp.float32),
            pltpu.VMEM((BH + H, TH), jnp.bfloat16),
            pltpu.SemaphoreType.DMA((3,)),
            pltpu.SemaphoreType.DMA((N_DEV,)),
            pltpu.SemaphoreType.DMA((N_DEV,)),
        ],
        compiler_params=pltpu.CompilerParams(collective_id=0),
    )(qp, k2, v2, cmat)


# device time: 15468 ns/iter; 1.4442x vs baseline; 1.0233x over previous
import numpy as np

import jax
import jax.numpy as jnp
from jax import lax
from jax.experimental import pallas as pl
from jax.experimental.pallas import tpu as pltpu

N_DEV = 16
B, H, D = 8, 8, 64
PAGES_PER_DEV = 64
BLOCK = 16
T_LOCAL = PAGES_PER_DEV * BLOCK
TH = T_LOCAL * H
BH = B * H
LANES = 128

_BH_HEAD = (np.arange(H)[:, None] == (np.arange(TH)[None, :] & 7)).astype(
    np.float32)


def kernel(Q, K, V, bt, lens):
    hbm = pltpu.MemorySpace.HBM
    k2 = pltpu.with_memory_space_constraint(K.reshape(TH, D), hbm)
    v2 = pltpu.with_memory_space_constraint(V.reshape(TH, D), hbm)
    q64 = pltpu.with_memory_space_constraint(Q.reshape(BH, D), hbm)
    cmat = pltpu.with_memory_space_constraint(
        jnp.asarray(_BH_HEAD, dtype=jnp.bfloat16), hbm)
    bt2 = pltpu.with_memory_space_constraint(
        jnp.concatenate(
            [bt, jnp.broadcast_to(lens[:, None], (B, PAGES_PER_DEV))], axis=0),
        hbm)

    def body(q_ref, k_ref, v_ref, c_ref, bt_ref, out_ref,
             comm_ref, kv_vmem, c_vmem, q_vmem, bt_vmem, res_vmem,
             dma_sems, send_sems, recv_sems):
        my = lax.axis_index("i")
        scope = jax.named_scope

        with scope("ph_dma_start"):
            btcopy = pltpu.make_async_copy(bt_ref, bt_vmem, dma_sems.at[0])
            ccopy = pltpu.make_async_copy(c_ref, c_vmem, dma_sems.at[1])
            qcopy = pltpu.make_async_copy(q_ref, q_vmem, dma_sems.at[2])
            kcopy = pltpu.make_async_copy(k_ref, kv_vmem.at[0], dma_sems.at[3])
            vcopy = pltpu.make_async_copy(v_ref, kv_vmem.at[1], dma_sems.at[4])
            btcopy.start()
            ccopy.start()
            qcopy.start()
            kcopy.start()
            vcopy.start()

        barrier = pltpu.get_barrier_semaphore()
        for o in range(1, N_DEV):
            peer = lax.rem(my + o, N_DEV)
            pl.semaphore_signal(
                barrier, inc=1, device_id=(peer,),
                device_id_type=pl.DeviceIdType.MESH,
            )

        with scope("ph_btdma_wait"):
            btcopy.wait()
        with scope("ph_wbuild"):
            base_f = lax.convert_element_type(my * PAGES_PER_DEV, jnp.float32)
            bti = bt_vmem[0:B, :]
            lensi = bt_vmem[B:2 * B, :]
            jl = lax.broadcasted_iota(jnp.int32, (B, PAGES_PER_DEV), 1)
            btm = jnp.where(jl < lensi, bti, -1).astype(jnp.float32)
            btm_t = btm.T
            pidf = base_f + lax.broadcasted_iota(
                jnp.int32, (PAGES_PER_DEV, PAGES_PER_DEV), 1
            ).astype(jnp.float32)
            rows = []
            for i in range(B):
                cmp = (btm_t[:, i:i + 1] == pidf).astype(jnp.float32)
                rows.append(jnp.sum(cmp, axis=0, keepdims=True))
            w_page = jnp.concatenate(rows, axis=0)

        with scope("ph_wexpand"):
            erep = (lax.broadcasted_iota(jnp.int32, (BH, B), 0) // H
                    == lax.broadcasted_iota(jnp.int32, (BH, B), 1)
                    ).astype(jnp.float32)
            wp64 = lax.dot_general(
                erep, w_page, (((1,), (0,)), ((), ())),
                preferred_element_type=jnp.float32,
            )
            wpb = jnp.repeat(wp64, TH // PAGES_PER_DEV, axis=1)
            ahead = (lax.broadcasted_iota(jnp.int32, (BH, H), 0) % H
                     == lax.broadcasted_iota(jnp.int32, (BH, H), 1)
                     ).astype(jnp.bfloat16)
            ccopy.wait()
            hm = lax.dot_general(
                ahead, c_vmem[:, :], (((1,), (0,)), ((), ())),
                preferred_element_type=jnp.float32,
            )
            wm_big = wpb * hm

        with scope("ph_kdma_wait"):
            qcopy.wait()
            kcopy.wait()
        with scope("ph_qk_softmax"):
            scale = jnp.float32(D ** -0.5)
            qflat = q_vmem[:, :] * scale
            s_big = lax.dot_general(
                qflat, kv_vmem[0], (((1,), (1,)), ((), ())),
                preferred_element_type=jnp.float32,
            )
            m = jnp.max(s_big, axis=1, keepdims=True)
            p = jnp.exp(s_big - m) * wm_big
            l = jnp.sum(p, axis=1, keepdims=True)
        with scope("ph_vdma_wait"):
            vcopy.wait()
        with scope("ph_pv"):
            o64 = lax.dot_general(
                p, kv_vmem[1], (((1,), (0,)), ((), ())),
                preferred_element_type=jnp.float32,
            )

        with scope("ph_pack"):
            comm_ref[0, :, 0:D] = o64.astype(jnp.bfloat16)
            comm_ref[0, :, D:D + 1] = m.astype(jnp.bfloat16)
            comm_ref[0, :, D + 1:D + 2] = l.astype(jnp.bfloat16)

        rdmas = {}
        with scope("ph_barrier"):
            pl.semaphore_wait(barrier, N_DEV - 1)
        with scope("ph_rdma_issue"):
            for o in range(1, N_DEV):
                target = lax.rem(my + o, N_DEV)
                slot = N_DEV - o
                rdmas[o] = pltpu.make_async_remote_copy(
                    src_ref=comm_ref.at[0],
                    dst_ref=comm_ref.at[slot],
                    send_sem=send_sems.at[o],
                    recv_sem=recv_sems.at[slot],
                    device_id=(target,),
                    device_id_type=pl.DeviceIdType.MESH,
                )
                rdmas[o].start()
        with scope("ph_rdma_wait"):
            for o in range(1, N_DEV):
                rdmas[o].wait()

        with scope("ph_merge"):
            o_all = comm_ref[:, :, 0:D].astype(jnp.float32)
            m_all = comm_ref[:, :, D:D + 1].astype(jnp.float32)
            l_all = comm_ref[:, :, D + 1:D + 2].astype(jnp.float32)
            m_g = jnp.max(m_all, axis=0)
            sc = jnp.exp(m_all - m_g[None])
            o_g = jnp.sum(o_all * sc, axis=0)
            l_g = jnp.sum(l_all * sc, axis=0)
            res = o_g / l_g

        with scope("ph_store"):
            for i in range(B):
                res_vmem[i, 0, :, :] = res[i * H:(i + 1) * H, :]
            outcopy = pltpu.make_async_copy(res_vmem, out_ref, dma_sems.at[5])
            outcopy.start()
            outcopy.wait()

    return pl.pallas_call(
        body,
        out_shape=jax.ShapeDtypeStruct((B, 1, H, D), jnp.float32),
        in_specs=[
            pl.BlockSpec(memory_space=pltpu.MemorySpace.HBM),
            pl.BlockSpec(memory_space=pltpu.MemorySpace.HBM),
            pl.BlockSpec(memory_space=pltpu.MemorySpace.HBM),
            pl.BlockSpec(memory_space=pltpu.MemorySpace.HBM),
            pl.BlockSpec(memory_space=pltpu.MemorySpace.HBM),
        ],
        out_specs=pl.BlockSpec(memory_space=pltpu.MemorySpace.HBM),
        scratch_shapes=[
            pltpu.VMEM((N_DEV, BH, LANES), jnp.bfloat16),
            pltpu.VMEM((2, TH, D), jnp.float32),
            pltpu.VMEM((H, TH), jnp.bfloat16),
            pltpu.VMEM((BH, D), jnp.float32),
            pltpu.VMEM((2 * B, PAGES_PER_DEV), jnp.int32),
            pltpu.VMEM((B, 1, H, D), jnp.float32),
            pltpu.SemaphoreType.DMA((6,)),
            pltpu.SemaphoreType.DMA((N_DEV,)),
            pltpu.SemaphoreType.DMA((N_DEV,)),
        ],
        compiler_params=pltpu.CompilerParams(collective_id=0),
    )(q64, k2, v2, cmat, bt2)


# device time: 13745 ns/iter; 1.6252x vs baseline; 1.1254x over previous
import numpy as np

import jax
import jax.numpy as jnp
from jax import lax
from jax.experimental import pallas as pl
from jax.experimental.pallas import tpu as pltpu

N_DEV = 16
B, H, D = 8, 8, 64
PAGES_PER_DEV = 64
BLOCK = 16
T_LOCAL = PAGES_PER_DEV * BLOCK
TH = T_LOCAL * H
BH = B * H
LANES = 128

_c = np.arange(TH)
_CONST = np.concatenate(
    [
        (np.arange(PAGES_PER_DEV)[:, None] == (_c[None, :] >> 7)),
        (np.arange(H)[:, None] == (_c[None, :] & 7)),
    ],
    axis=0,
).astype(np.float32)


def kernel(Q, K, V, bt, lens):
    hbm = pltpu.MemorySpace.HBM
    k2 = pltpu.with_memory_space_constraint(K.reshape(TH, D), hbm)
    v2 = pltpu.with_memory_space_constraint(V.reshape(TH, D), hbm)
    q64 = pltpu.with_memory_space_constraint(Q.reshape(BH, D), hbm)
    cmat = pltpu.with_memory_space_constraint(
        jnp.asarray(_CONST, dtype=jnp.bfloat16), hbm)
    btc = pltpu.with_memory_space_constraint(bt, hbm)
    lens1 = pltpu.with_memory_space_constraint(
        lens.reshape(1, B), hbm)

    def body(q_ref, k_ref, v_ref, c_ref, bt_ref, lens_ref, out_ref,
             comm_ref, kv_vmem, c_vmem, q_vmem, bt_vmem, lens_vmem, res_vmem,
             dma_sems, send_sems, recv_sems):
        my = lax.axis_index("i")
        scope = jax.named_scope

        with scope("ph_dma_start"):
            btcopy = pltpu.make_async_copy(bt_ref, bt_vmem, dma_sems.at[0])
            lcopy = pltpu.make_async_copy(lens_ref, lens_vmem, dma_sems.at[5])
            ccopy = pltpu.make_async_copy(c_ref, c_vmem, dma_sems.at[1])
            qcopy = pltpu.make_async_copy(q_ref, q_vmem, dma_sems.at[2])
            kcopy = pltpu.make_async_copy(k_ref, kv_vmem.at[0], dma_sems.at[3])
            vcopy = pltpu.make_async_copy(v_ref, kv_vmem.at[1], dma_sems.at[4])
            btcopy.start()
            lcopy.start()
            ccopy.start()
            qcopy.start()
            kcopy.start()
            vcopy.start()

        barrier = pltpu.get_barrier_semaphore()
        for o in range(1, N_DEV):
            peer = lax.rem(my + o, N_DEV)
            pl.semaphore_signal(
                barrier, inc=1, device_id=(peer,),
                device_id_type=pl.DeviceIdType.MESH,
            )

        with scope("ph_btdma_wait"):
            btcopy.wait()
            lcopy.wait()
        with scope("ph_wbuild"):
            base_f = lax.convert_element_type(my * PAGES_PER_DEV, jnp.float32)
            btf_t = bt_vmem[:, :].astype(jnp.float32).T
            lens_f = lens_vmem[:, :].astype(jnp.float32)
            jt = lax.broadcasted_iota(
                jnp.int32, (PAGES_PER_DEV, B), 0).astype(jnp.float32)
            btm_t = jnp.where(jt < lens_f, btf_t, -1.0)
            pidf = base_f + lax.broadcasted_iota(
                jnp.int32, (PAGES_PER_DEV, PAGES_PER_DEV), 1
            ).astype(jnp.float32)
            rows = []
            for i in range(B):
                cmp = (btm_t[:, i:i + 1] == pidf).astype(jnp.float32)
                rows.append(jnp.sum(cmp, axis=0, keepdims=True))
            w_page = jnp.concatenate(rows, axis=0)

        with scope("ph_wexpand"):
            erep = (lax.broadcasted_iota(jnp.int32, (BH, B), 0) // H
                    == lax.broadcasted_iota(jnp.int32, (BH, B), 1)
                    ).astype(jnp.float32)
            wp64 = lax.dot_general(
                erep, w_page, (((1,), (0,)), ((), ())),
                preferred_element_type=jnp.float32,
            )
            ccopy.wait()
            wpb = lax.dot_general(
                wp64.astype(jnp.bfloat16), c_vmem[0:PAGES_PER_DEV, :],
                (((1,), (0,)), ((), ())),
                preferred_element_type=jnp.float32,
            )
            ahead = (lax.broadcasted_iota(jnp.int32, (BH, H), 0) % H
                     == lax.broadcasted_iota(jnp.int32, (BH, H), 1)
                     ).astype(jnp.bfloat16)
            hm = lax.dot_general(
                ahead, c_vmem[PAGES_PER_DEV:PAGES_PER_DEV + H, :],
                (((1,), (0,)), ((), ())),
                preferred_element_type=jnp.float32,
            )
            wm_big = wpb * hm

        with scope("ph_kdma_wait"):
            qcopy.wait()
            kcopy.wait()
        with scope("ph_qk_softmax"):
            scale = jnp.float32(D ** -0.5)
            qflat = q_vmem[:, :] * scale
            s_big = lax.dot_general(
                qflat, kv_vmem[0], (((1,), (1,)), ((), ())),
                preferred_element_type=jnp.float32,
            )
            m = jnp.max(s_big, axis=1, keepdims=True)
            p = jnp.exp(s_big - m) * wm_big
            l = jnp.sum(p, axis=1, keepdims=True)
        with scope("ph_vdma_wait"):
            vcopy.wait()
        with scope("ph_pv"):
            o64 = lax.dot_general(
                p, kv_vmem[1], (((1,), (0,)), ((), ())),
                preferred_element_type=jnp.float32,
            )

        with scope("ph_pack"):
            comm_ref[0, :, 0:D] = o64.astype(jnp.bfloat16)
            comm_ref[0, :, D:D + 1] = m.astype(jnp.bfloat16)
            comm_ref[0, :, D + 1:D + 2] = l.astype(jnp.bfloat16)

        rdmas = {}
        with scope("ph_barrier"):
            pl.semaphore_wait(barrier, N_DEV - 1)
        with scope("ph_rdma_issue"):
            for o in range(1, N_DEV):
                target = lax.rem(my + o, N_DEV)
                slot = N_DEV - o
                rdmas[o] = pltpu.make_async_remote_copy(
                    src_ref=comm_ref.at[0],
                    dst_ref=comm_ref.at[slot],
                    send_sem=send_sems.at[o],
                    recv_sem=recv_sems.at[slot],
                    device_id=(target,),
                    device_id_type=pl.DeviceIdType.MESH,
                )
                rdmas[o].start()
        def lse_partial(lo, hi):
            o_a = comm_ref[lo:hi, :, 0:D].astype(jnp.float32)
            m_a = comm_ref[lo:hi, :, D:D + 1].astype(jnp.float32)
            l_a = comm_ref[lo:hi, :, D + 1:D + 2].astype(jnp.float32)
            m_p = jnp.max(m_a, axis=0)
            sc = jnp.exp(m_a - m_p[None])
            return (jnp.sum(o_a * sc, axis=0), jnp.sum(l_a * sc, axis=0), m_p)

        with scope("ph_rdma_wait1"):
            for o in range(1, 9):
                rdmas[o].wait()
        with scope("ph_merge1"):
            o1, l1, m1 = lse_partial(8, N_DEV)
        with scope("ph_rdma_wait2"):
            for o in range(9, N_DEV):
                rdmas[o].wait()
        with scope("ph_merge2"):
            o2, l2, m2 = lse_partial(0, 8)
            m_g = jnp.maximum(m1, m2)
            s1 = jnp.exp(m1 - m_g)
            s2 = jnp.exp(m2 - m_g)
            o_g = o1 * s1 + o2 * s2
            l_g = l1 * s1 + l2 * s2
            res = o_g / l_g

        with scope("ph_store"):
            for i in range(B):
                res_vmem[i, 0, :, :] = res[i * H:(i + 1) * H, :]
            outcopy = pltpu.make_async_copy(res_vmem, out_ref, dma_sems.at[6])
            outcopy.start()
            outcopy.wait()

    return pl.pallas_call(
        body,
        out_shape=jax.ShapeDtypeStruct((B, 1, H, D), jnp.float32),
        in_specs=[
            pl.BlockSpec(memory_space=pltpu.MemorySpace.HBM),
            pl.BlockSpec(memory_space=pltpu.MemorySpace.HBM),
            pl.BlockSpec(memory_space=pltpu.MemorySpace.HBM),
            pl.BlockSpec(memory_space=pltpu.MemorySpace.HBM),
            pl.BlockSpec(memory_space=pltpu.MemorySpace.HBM),
            pl.BlockSpec(memory_space=pltpu.MemorySpace.HBM),
        ],
        out_specs=pl.BlockSpec(memory_space=pltpu.MemorySpace.HBM),
        scratch_shapes=[
            pltpu.VMEM((N_DEV, BH, LANES), jnp.bfloat16),
            pltpu.VMEM((2, TH, D), jnp.float32),
            pltpu.VMEM((BH + H, TH), jnp.bfloat16),
            pltpu.VMEM((BH, D), jnp.float32),
            pltpu.VMEM((B, PAGES_PER_DEV), jnp.int32),
            pltpu.VMEM((1, B), jnp.int32),
            pltpu.VMEM((B, 1, H, D), jnp.float32),
            pltpu.SemaphoreType.DMA((7,)),
            pltpu.SemaphoreType.DMA((N_DEV,)),
            pltpu.SemaphoreType.DMA((N_DEV,)),
        ],
        compiler_params=pltpu.CompilerParams(collective_id=0),
    )(q64, k2, v2, cmat, btc, lens1)
